# baseline (speedup 1.0000x reference)
import functools

import jax
import jax.numpy as jnp
from jax import lax
from jax.experimental import pallas as pl
from jax.experimental.pallas import tpu as pltpu
from jax.experimental.pallas import tpu_sc as plsc

_NW = 32
_IPR = 128
_EPS = 1e-5


def _rsqrt16(x):
    i = plsc.bitcast(x, jnp.int32)
    i = jnp.int32(0x5F3759DF) - (i >> 1)
    y = plsc.bitcast(i, jnp.float32)
    xh = x * 0.5
    for _ in range(3):
        y = y * (1.5 - xh * y * y)
    return y


def kernel(input_ids, word_table, pos_table, gamma, beta):
    B, L = input_ids.shape
    V, D = word_table.shape
    N = B * L
    C = 512
    RW = N // _NW
    NCH = RW // C

    ids3d = input_ids.reshape(N // C, C // _IPR, _IPR).astype(jnp.int32)

    mesh = plsc.VectorSubcoreMesh(core_axis_name="c", subcore_axis_name="s")

    @functools.partial(
        pl.kernel,
        mesh=mesh,
        out_type=jax.ShapeDtypeStruct((N, D), jnp.float32),
        compiler_params=pltpu.CompilerParams(
            needs_layout_passes=False, use_tc_tiling_on_sc=False),
        scratch_types=[
            pltpu.VMEM((C // _IPR, _IPR), jnp.int32),
            pltpu.VMEM((C, D), jnp.float32),
            pltpu.VMEM((L, D), jnp.float32),
            pltpu.VMEM((D,), jnp.float32),
            pltpu.VMEM((D,), jnp.float32),
            pltpu.SemaphoreType.DMA,
        ],
    )
    def _k(ids_hbm, word_hbm, pos_hbm, gamma_hbm, beta_hbm, out_hbm,
           idx_v, rows_v, pos_v, gamma_v, beta_v, sem):
        wid = lax.axis_index("s") * 2 + lax.axis_index("c")
        pltpu.sync_copy(pos_hbm.at[pl.ds(0, L)], pos_v)
        pltpu.sync_copy(gamma_hbm, gamma_v)
        pltpu.sync_copy(beta_hbm, beta_v)
        g = [gamma_v[pl.ds(16 * k, 16)] for k in range(4)]
        b = [beta_v[pl.ds(16 * k, 16)] for k in range(4)]

        def chunk_body(c, carry):
            base = wid * RW + c * C
            pltpu.sync_copy(ids_hbm.at[wid * NCH + c], idx_v)
            cps = [
                pltpu.async_copy(
                    word_hbm.at[idx_v.at[j]],
                    rows_v.at[pl.ds(j * _IPR, _IPR)], sem)
                for j in range(C // _IPR)
            ]
            for cp in cps:
                cp.wait()

            def row_body(r, rcarry):
                l = (base + r) % L
                x = [rows_v[r, pl.ds(16 * k, 16)] + pos_v[l, pl.ds(16 * k, 16)]
                     for k in range(4)]
                s = (x[0] + x[1]) + (x[2] + x[3])
                mean = jnp.broadcast_to(jnp.sum(s), (16,)) * (1.0 / D)
                d = [xx - mean for xx in x]
                q = (d[0] * d[0] + d[1] * d[1]) + (d[2] * d[2] + d[3] * d[3])
                var = jnp.broadcast_to(jnp.sum(q), (16,)) * (1.0 / D)
                rstd = _rsqrt16(var + _EPS)
                for k in range(4):
                    rows_v[r, pl.ds(16 * k, 16)] = d[k] * rstd * g[k] + b[k]
                return rcarry

            lax.fori_loop(0, C, row_body, 0)
            pltpu.sync_copy(rows_v, out_hbm.at[pl.ds(base, C)])
            return carry

        lax.fori_loop(0, NCH, chunk_body, 0)

    out = _k(ids3d, word_table, pos_table, gamma, beta)
    return out.reshape(B, L, D)
